# Initial kernel scaffold; baseline (speedup 1.0000x reference)
import jax
import jax.numpy as jnp
from jax.experimental import pallas as pl


def kernel(positions, hidden_states, q_c, kv_lora, Wq_b, kv_a_ln_w, Wkv_b, Wo, Wq_idx, Wk_idx, k_norm_w, k_norm_b, Wweights):
    raise NotImplementedError("write your pallas kernel here")


# speedup vs baseline: 7.5121x; 7.5121x over previous
import jax
import jax.numpy as jnp
from jax.experimental import pallas as pl
from jax.experimental.pallas import tpu as pltpu

T = 2048
HID = 2048
H = 16
NOPE = 128
ROPE = 64
VD = 128
QL = 1536
KVR = 512
IH = 8
ID = 128
TOPK = 512
EPS = 1e-6

BT = 256
BTQ = 128
BTK = 512
SCALING = (NOPE + ROPE) ** -0.5
IDX_SCALE = (ID ** -0.5) * (IH ** -0.5)

_INTERPRET = False


def _rope_neox(x, cos, sin):
    half = ROPE // 2
    x1 = x[:, :half]
    x2 = x[:, half:]
    return jnp.concatenate([x1 * cos - x2 * sin, x2 * cos + x1 * sin], axis=1)


def _prep_kernel(hs_ref, kvc_ref, kpe_ref, cos_ref, sin_ref, wk_ref, ww_ref,
                 knw_ref, knb_ref, kvln_ref,
                 kiT_ref, wts_ref, kfull_ref, kfullT_ref):
    hs = hs_ref[...]
    cos = cos_ref[...]
    sin = sin_ref[...]
    ki = jnp.dot(hs, wk_ref[...], preferred_element_type=jnp.float32)
    mu = jnp.mean(ki, axis=1, keepdims=True)
    var = jnp.mean((ki - mu) ** 2, axis=1, keepdims=True)
    ki = (ki - mu) * jax.lax.rsqrt(var + 1e-6) * knw_ref[...] + knb_ref[...]
    ki_full = jnp.concatenate([_rope_neox(ki[:, :ROPE], cos, sin), ki[:, ROPE:]], axis=1)
    kiT_ref[...] = ki_full.astype(jnp.bfloat16).T
    wts_ref[...] = jnp.dot(hs, ww_ref[...], preferred_element_type=jnp.float32) * IDX_SCALE
    kvc = kvc_ref[...]
    var2 = jnp.mean(kvc * kvc, axis=1, keepdims=True)
    kvn = kvc * jax.lax.rsqrt(var2 + EPS) * kvln_ref[...]
    kf = jnp.concatenate([kvn, _rope_neox(kpe_ref[...], cos, sin)], axis=1).astype(jnp.bfloat16)
    kfull_ref[...] = kf
    kfullT_ref[...] = kf.T


def _indexer_kernel(qc_ref, wqi_ref, cos_ref, sin_ref, kiT_ref, wts_ref,
                    us_ref, thr_ref, relu_ref):
    tq = pl.program_id(0)
    qi = jnp.dot(qc_ref[...], wqi_ref[...], preferred_element_type=jnp.float32)
    cos = cos_ref[...]
    sin = sin_ref[...]
    kiT = kiT_ref[...]
    wts = wts_ref[...]
    for h in range(IH):
        b = h * ID
        pe = _rope_neox(qi[:, b:b + ROPE], cos, sin)
        qih = jnp.concatenate([pe, qi[:, b + ROPE:b + ID]], axis=1).astype(jnp.bfloat16)
        s = jnp.dot(qih, kiT, preferred_element_type=jnp.float32).astype(jnp.bfloat16)
        relu_ref[h * BT:(h + 1) * BT, :] = jnp.maximum(s, jnp.bfloat16(0))
    row = jax.lax.broadcasted_iota(jnp.int32, (BT, IH * BT), 0)
    colh = jax.lax.broadcasted_iota(jnp.int32, (BT, IH * BT), 1)
    wfull = jnp.zeros((BT, IH * BT), jnp.float32)
    for h in range(IH):
        wfull = jnp.where(colh // BT == h, wts[:, h:h + 1], wfull)
    amat = jnp.where(colh % BT == row, wfull, 0.0).astype(jnp.bfloat16)
    acc = jnp.dot(amat, relu_ref[...], preferred_element_type=jnp.float32)
    row = tq * BT + jax.lax.broadcasted_iota(jnp.int32, (BT, T), 0)
    col = jax.lax.broadcasted_iota(jnp.int32, (BT, T), 1)
    scores = jnp.where(col <= row, acc, -1e30)
    b32 = jax.lax.bitcast_convert_type(scores, jnp.uint32)
    us = jnp.where(b32 >> 31 != 0, ~b32, b32 | jnp.uint32(0x80000000))
    us_ref[...] = us
    prefix = jnp.zeros((BT, 1), jnp.uint32)
    for bit in range(31, -1, -1):
        cand = prefix | jnp.uint32(1 << bit)
        cnt = jnp.sum((us >= cand).astype(jnp.int32), axis=1, keepdims=True)
        prefix = jnp.where(cnt >= TOPK, cand, prefix)
    thr_ref[...] = prefix


def _qproj_kernel(qc_ref, wqb_ref, wukT_ref, cos_ref, sin_ref, qfull_ref):
    q = jnp.dot(qc_ref[...], wqb_ref[0], preferred_element_type=jnp.float32)
    pe = _rope_neox(q[:, NOPE:], cos_ref[...], sin_ref[...])
    qlat = jnp.dot(q[:, :NOPE], wukT_ref[0], preferred_element_type=jnp.float32)
    qfull_ref[0] = jnp.concatenate([qlat, pe], axis=1).astype(jnp.bfloat16)


def _flash_kernel(qfull_ref, kT_ref, kfull_ref, us_ref, thr_ref, wuv_ref,
                  v_ref, acc_ref, m_ref, l_ref):
    tq = pl.program_id(0)
    kk = pl.program_id(1)

    @pl.when(kk == 0)
    def _init():
        m_ref[...] = jnp.full_like(m_ref, -1e30)
        l_ref[...] = jnp.zeros_like(l_ref)
        acc_ref[...] = jnp.zeros_like(acc_ref)

    @pl.when(kk * BTK <= tq * BTQ + (BTQ - 1))
    def _compute():
        kT = kT_ref[...]
        vblk = kfull_ref[...][:, :KVR]
        row = tq * BTQ + jax.lax.broadcasted_iota(jnp.int32, (BTQ, BTK), 0)
        col = kk * BTK + jax.lax.broadcasted_iota(jnp.int32, (BTQ, BTK), 1)
        mask = (us_ref[...] >= thr_ref[...]) & (col <= row)
        for h in range(H):
            s = jnp.dot(qfull_ref[h], kT, preferred_element_type=jnp.float32) * SCALING
            s = jnp.where(mask, s, -1e30)
            m_old = m_ref[h]
            m_new = jnp.maximum(m_old, jnp.max(s, axis=1, keepdims=True))
            alpha = jnp.exp(m_old - m_new)
            p = jnp.where(mask, jnp.exp(s - m_new), 0.0)
            l_ref[h] = l_ref[h] * alpha + jnp.sum(p, axis=1, keepdims=True)
            acc_ref[h] = acc_ref[h] * alpha + jnp.dot(p.astype(jnp.bfloat16), vblk, preferred_element_type=jnp.float32)
            m_ref[h] = m_new

    @pl.when(kk == (tq * BTQ + (BTQ - 1)) // BTK)
    def _finalize():
        for h in range(H):
            attn = (acc_ref[h] / l_ref[h]).astype(jnp.bfloat16)
            v_ref[:, h * VD:(h + 1) * VD] = jnp.dot(
                attn, wuv_ref[h], preferred_element_type=jnp.float32)


def _oproj_kernel(v_ref, wo_ref, out_ref):
    out_ref[...] = jnp.dot(v_ref[...], wo_ref[...], preferred_element_type=jnp.float32)


def kernel(positions, hidden_states, q_c, kv_lora, Wq_b, kv_a_ln_w, Wkv_b, Wo,
           Wq_idx, Wk_idx, k_norm_w, k_norm_b, Wweights):
    f32 = jnp.float32
    inv = 1.0 / (10000.0 ** (jnp.arange(0, ROPE, 2, dtype=f32) / ROPE))
    ang = positions.astype(f32)[:, None] * inv[None, :]
    cos = jnp.cos(ang)
    sin = jnp.sin(ang)
    kv_c = kv_lora[:, :KVR]
    perm = jnp.concatenate([jnp.arange(0, ROPE, 2), jnp.arange(1, ROPE, 2)])
    k_pe = kv_lora[:, KVR:][:, perm]
    Wq_b_r = Wq_b.reshape(QL, H, NOPE + ROPE)
    Wq_b_r = jnp.concatenate(
        [Wq_b_r[:, :, :NOPE], Wq_b_r[:, :, NOPE:][:, :, perm]], axis=2)
    Wq_b_r = Wq_b_r.transpose(1, 0, 2)
    Wkvb = Wkv_b.reshape(KVR, H, NOPE + VD)
    W_UK_T = Wkvb[:, :, :NOPE].transpose(1, 2, 0)
    W_UV = Wkvb[:, :, NOPE:].transpose(1, 0, 2)
    bf16 = jnp.bfloat16
    W_UVb = W_UV.astype(bf16)
    knw = k_norm_w.reshape(1, ID)
    knb = k_norm_b.reshape(1, ID)
    kvln = kv_a_ln_w.reshape(1, KVR)

    nb = T // BT

    kiT, wts, k_full, k_fullT = pl.pallas_call(
        _prep_kernel,
        grid=(nb,),
        in_specs=[
            pl.BlockSpec((BT, HID), lambda i: (i, 0)),
            pl.BlockSpec((BT, KVR), lambda i: (i, 0)),
            pl.BlockSpec((BT, ROPE), lambda i: (i, 0)),
            pl.BlockSpec((BT, ROPE // 2), lambda i: (i, 0)),
            pl.BlockSpec((BT, ROPE // 2), lambda i: (i, 0)),
            pl.BlockSpec((HID, ID), lambda i: (0, 0)),
            pl.BlockSpec((HID, IH), lambda i: (0, 0)),
            pl.BlockSpec((1, ID), lambda i: (0, 0)),
            pl.BlockSpec((1, ID), lambda i: (0, 0)),
            pl.BlockSpec((1, KVR), lambda i: (0, 0)),
        ],
        out_specs=[
            pl.BlockSpec((ID, BT), lambda i: (0, i)),
            pl.BlockSpec((BT, IH), lambda i: (i, 0)),
            pl.BlockSpec((BT, KVR + ROPE), lambda i: (i, 0)),
            pl.BlockSpec((KVR + ROPE, BT), lambda i: (0, i)),
        ],
        out_shape=[
            jax.ShapeDtypeStruct((ID, T), jnp.bfloat16),
            jax.ShapeDtypeStruct((T, IH), f32),
            jax.ShapeDtypeStruct((T, KVR + ROPE), jnp.bfloat16),
            jax.ShapeDtypeStruct((KVR + ROPE, T), jnp.bfloat16),
        ],
        interpret=_INTERPRET,
    )(hidden_states, kv_c, k_pe, cos, sin, Wk_idx, Wweights, knw, knb, kvln)

    us, thr = pl.pallas_call(
        _indexer_kernel,
        grid=(nb,),
        in_specs=[
            pl.BlockSpec((BT, QL), lambda i: (i, 0)),
            pl.BlockSpec((QL, IH * ID), lambda i: (0, 0)),
            pl.BlockSpec((BT, ROPE // 2), lambda i: (i, 0)),
            pl.BlockSpec((BT, ROPE // 2), lambda i: (i, 0)),
            pl.BlockSpec((ID, T), lambda i: (0, 0)),
            pl.BlockSpec((BT, IH), lambda i: (i, 0)),
        ],
        out_specs=[
            pl.BlockSpec((BT, T), lambda i: (i, 0)),
            pl.BlockSpec((BT, 1), lambda i: (i, 0)),
        ],
        out_shape=[
            jax.ShapeDtypeStruct((T, T), jnp.uint32),
            jax.ShapeDtypeStruct((T, 1), jnp.uint32),
        ],
        scratch_shapes=[pltpu.VMEM((IH * BT, T), jnp.bfloat16)],
        interpret=_INTERPRET,
    )(q_c, Wq_idx, cos, sin, kiT, wts)

    q_full = pl.pallas_call(
        _qproj_kernel,
        grid=(nb, H),
        in_specs=[
            pl.BlockSpec((BT, QL), lambda i, h: (i, 0)),
            pl.BlockSpec((1, QL, NOPE + ROPE), lambda i, h: (h, 0, 0)),
            pl.BlockSpec((1, NOPE, KVR), lambda i, h: (h, 0, 0)),
            pl.BlockSpec((BT, ROPE // 2), lambda i, h: (i, 0)),
            pl.BlockSpec((BT, ROPE // 2), lambda i, h: (i, 0)),
        ],
        out_specs=pl.BlockSpec((1, BT, KVR + ROPE), lambda i, h: (h, i, 0)),
        out_shape=jax.ShapeDtypeStruct((H, T, KVR + ROPE), jnp.bfloat16),
        interpret=_INTERPRET,
    )(q_c, Wq_b_r, W_UK_T, cos, sin)

    v = pl.pallas_call(
        _flash_kernel,
        grid=(T // BTQ, T // BTK),
        in_specs=[
            pl.BlockSpec((H, BTQ, KVR + ROPE), lambda i, k: (0, i, 0)),
            pl.BlockSpec((KVR + ROPE, BTK), lambda i, k: (0, k)),
            pl.BlockSpec((BTK, KVR + ROPE), lambda i, k: (k, 0)),
            pl.BlockSpec((BTQ, BTK), lambda i, k: (i, k)),
            pl.BlockSpec((BTQ, 1), lambda i, k: (i, 0)),
            pl.BlockSpec((H, KVR, VD), lambda i, k: (0, 0, 0)),
        ],
        out_specs=pl.BlockSpec((BTQ, H * VD), lambda i, k: (i, 0)),
        out_shape=jax.ShapeDtypeStruct((T, H * VD), f32),
        scratch_shapes=[
            pltpu.VMEM((H, BTQ, KVR), f32),
            pltpu.VMEM((H, BTQ, 1), f32),
            pltpu.VMEM((H, BTQ, 1), f32),
        ],
        interpret=_INTERPRET,
    )(q_full, k_fullT, k_full, us, thr, W_UVb)

    out = pl.pallas_call(
        _oproj_kernel,
        grid=(nb,),
        in_specs=[
            pl.BlockSpec((BT, H * VD), lambda i: (i, 0)),
            pl.BlockSpec((H * VD, HID), lambda i: (0, 0)),
        ],
        out_specs=pl.BlockSpec((BT, HID), lambda i: (i, 0)),
        out_shape=jax.ShapeDtypeStruct((T, HID), f32),
        interpret=_INTERPRET,
    )(v, Wo)

    return out


# speedup vs baseline: 8.8094x; 1.1727x over previous
import jax
import jax.numpy as jnp
from jax.experimental import pallas as pl
from jax.experimental.pallas import tpu as pltpu

T = 2048
HID = 2048
H = 16
NOPE = 128
ROPE = 64
VD = 128
QL = 1536
KVR = 512
IH = 8
ID = 128
TOPK = 512
EPS = 1e-6

BT = 256
BTQ = 256
BTK = 512
SCALING = (NOPE + ROPE) ** -0.5
IDX_SCALE = (ID ** -0.5) * (IH ** -0.5)

def _rope_neox(x, cos, sin):
    half = ROPE // 2
    x1 = x[:, :half]
    x2 = x[:, half:]
    return jnp.concatenate([x1 * cos - x2 * sin, x2 * cos + x1 * sin], axis=1)


def _prep_kernel(hs_ref, kvc_ref, kpe_ref, cos_ref, sin_ref, wk_ref, ww_ref,
                 knw_ref, knb_ref, kvln_ref,
                 kiT_ref, wts_ref, kfull_ref, kfullT_ref):
    hs = hs_ref[...]
    cos = cos_ref[...]
    sin = sin_ref[...]
    ki = jnp.dot(hs, wk_ref[...], preferred_element_type=jnp.float32)
    mu = jnp.mean(ki, axis=1, keepdims=True)
    var = jnp.mean((ki - mu) ** 2, axis=1, keepdims=True)
    ki = (ki - mu) * jax.lax.rsqrt(var + 1e-6) * knw_ref[...] + knb_ref[...]
    ki_full = jnp.concatenate([_rope_neox(ki[:, :ROPE], cos, sin), ki[:, ROPE:]], axis=1)
    kiT_ref[...] = ki_full.astype(jnp.bfloat16).T
    wts_ref[...] = jnp.dot(hs, ww_ref[...], preferred_element_type=jnp.float32) * IDX_SCALE
    kvc = kvc_ref[...]
    var2 = jnp.mean(kvc * kvc, axis=1, keepdims=True)
    kvn = kvc * jax.lax.rsqrt(var2 + EPS) * kvln_ref[...]
    kf = jnp.concatenate([kvn, _rope_neox(kpe_ref[...], cos, sin)], axis=1).astype(jnp.bfloat16)
    kfull_ref[...] = kf
    kfullT_ref[...] = kf.T


def _indexer_kernel(qc_ref, wqi_ref, cos_ref, sin_ref, kiT_ref, wts_ref,
                    us_ref, thr_ref, relu_ref):
    tq = pl.program_id(0)
    qi = jnp.dot(qc_ref[...], wqi_ref[...], preferred_element_type=jnp.float32)
    cos = cos_ref[...]
    sin = sin_ref[...]
    kiT = kiT_ref[...]
    wts = wts_ref[...]
    for h in range(IH):
        b = h * ID
        pe = _rope_neox(qi[:, b:b + ROPE], cos, sin)
        qih = jnp.concatenate([pe, qi[:, b + ROPE:b + ID]], axis=1).astype(jnp.bfloat16)
        s = jnp.dot(qih, kiT, preferred_element_type=jnp.float32).astype(jnp.bfloat16)
        relu_ref[h * BT:(h + 1) * BT, :] = jnp.maximum(s, jnp.bfloat16(0))
    row = jax.lax.broadcasted_iota(jnp.int32, (BT, IH * BT), 0)
    colh = jax.lax.broadcasted_iota(jnp.int32, (BT, IH * BT), 1)
    wfull = jnp.zeros((BT, IH * BT), jnp.float32)
    for h in range(IH):
        wfull = jnp.where(colh // BT == h, wts[:, h:h + 1], wfull)
    amat = jnp.where(colh % BT == row, wfull, 0.0).astype(jnp.bfloat16)
    acc = jnp.dot(amat, relu_ref[...], preferred_element_type=jnp.float32)
    row = tq * BT + jax.lax.broadcasted_iota(jnp.int32, (BT, T), 0)
    col = jax.lax.broadcasted_iota(jnp.int32, (BT, T), 1)
    scores = jnp.where(col <= row, acc, -1e30)
    b32 = jax.lax.bitcast_convert_type(scores, jnp.uint32)
    us = jnp.where(b32 >> 31 != 0, ~b32, b32 | jnp.uint32(0x80000000))
    us_ref[...] = us
    prefix = jnp.zeros((BT, 1), jnp.uint32)
    for bit in range(31, -1, -1):
        cand = prefix | jnp.uint32(1 << bit)
        cnt = jnp.sum((us >= cand).astype(jnp.int32), axis=1, keepdims=True)
        prefix = jnp.where(cnt >= TOPK, cand, prefix)
    thr_ref[...] = prefix


def _qproj_kernel(qc_ref, wqb_ref, wukT_ref, cos_ref, sin_ref, qfull_ref):
    q = jnp.dot(qc_ref[...], wqb_ref[0], preferred_element_type=jnp.float32)
    pe = _rope_neox(q[:, NOPE:], cos_ref[...], sin_ref[...])
    qlat = jnp.dot(q[:, :NOPE], wukT_ref[0], preferred_element_type=jnp.float32)
    qfull_ref[0] = jnp.concatenate([qlat, pe], axis=1).astype(jnp.bfloat16)


def _flash_kernel(qfull_ref, kT_ref, kfull_ref, us_ref, thr_ref, wuv_ref,
                  v_ref, acc_ref, m_ref, l_ref):
    tq = pl.program_id(0)
    kk = pl.program_id(1)

    @pl.when(kk == 0)
    def _init():
        m_ref[...] = jnp.full_like(m_ref, -1e30)
        l_ref[...] = jnp.zeros_like(l_ref)
        acc_ref[...] = jnp.zeros_like(acc_ref)

    @pl.when(kk * BTK <= tq * BTQ + (BTQ - 1))
    def _compute():
        kT = kT_ref[...]
        vblk = kfull_ref[...][:, :KVR]
        row = tq * BTQ + jax.lax.broadcasted_iota(jnp.int32, (BTQ, BTK), 0)
        col = kk * BTK + jax.lax.broadcasted_iota(jnp.int32, (BTQ, BTK), 1)
        mask = (us_ref[...] >= thr_ref[...]) & (col <= row)
        for h in range(H):
            s = jnp.dot(qfull_ref[h], kT, preferred_element_type=jnp.float32) * SCALING
            s = jnp.where(mask, s, -1e30)
            m_old = m_ref[h]
            m_new = jnp.maximum(m_old, jnp.max(s, axis=1, keepdims=True))
            alpha = jnp.exp(m_old - m_new)
            p = jnp.where(mask, jnp.exp(s - m_new), 0.0)
            l_ref[h] = l_ref[h] * alpha + jnp.sum(p, axis=1, keepdims=True)
            acc_ref[h] = acc_ref[h] * alpha + jnp.dot(p.astype(jnp.bfloat16), vblk, preferred_element_type=jnp.float32)
            m_ref[h] = m_new

    @pl.when(kk == (tq * BTQ + (BTQ - 1)) // BTK)
    def _finalize():
        for h in range(H):
            attn = (acc_ref[h] / l_ref[h]).astype(jnp.bfloat16)
            v_ref[:, h * VD:(h + 1) * VD] = jnp.dot(
                attn, wuv_ref[h], preferred_element_type=jnp.float32)


def _oproj_kernel(v_ref, wo_ref, out_ref):
    out_ref[...] = jnp.dot(v_ref[...], wo_ref[...], preferred_element_type=jnp.float32)


def kernel(positions, hidden_states, q_c, kv_lora, Wq_b, kv_a_ln_w, Wkv_b, Wo,
           Wq_idx, Wk_idx, k_norm_w, k_norm_b, Wweights):
    f32 = jnp.float32
    inv = 1.0 / (10000.0 ** (jnp.arange(0, ROPE, 2, dtype=f32) / ROPE))
    ang = positions.astype(f32)[:, None] * inv[None, :]
    cos = jnp.cos(ang)
    sin = jnp.sin(ang)
    kv_c = kv_lora[:, :KVR]
    perm = jnp.concatenate([jnp.arange(0, ROPE, 2), jnp.arange(1, ROPE, 2)])
    k_pe = kv_lora[:, KVR:][:, perm]
    Wq_b_r = Wq_b.reshape(QL, H, NOPE + ROPE)
    Wq_b_r = jnp.concatenate(
        [Wq_b_r[:, :, :NOPE], Wq_b_r[:, :, NOPE:][:, :, perm]], axis=2)
    Wq_b_r = Wq_b_r.transpose(1, 0, 2)
    Wkvb = Wkv_b.reshape(KVR, H, NOPE + VD)
    W_UK_T = Wkvb[:, :, :NOPE].transpose(1, 2, 0)
    W_UV = Wkvb[:, :, NOPE:].transpose(1, 0, 2)
    bf16 = jnp.bfloat16
    W_UVb = W_UV.astype(bf16)
    knw = k_norm_w.reshape(1, ID)
    knb = k_norm_b.reshape(1, ID)
    kvln = kv_a_ln_w.reshape(1, KVR)

    nb = T // BT

    kiT, wts, k_full, k_fullT = pl.pallas_call(
        _prep_kernel,
        grid=(nb,),
        in_specs=[
            pl.BlockSpec((BT, HID), lambda i: (i, 0)),
            pl.BlockSpec((BT, KVR), lambda i: (i, 0)),
            pl.BlockSpec((BT, ROPE), lambda i: (i, 0)),
            pl.BlockSpec((BT, ROPE // 2), lambda i: (i, 0)),
            pl.BlockSpec((BT, ROPE // 2), lambda i: (i, 0)),
            pl.BlockSpec((HID, ID), lambda i: (0, 0)),
            pl.BlockSpec((HID, IH), lambda i: (0, 0)),
            pl.BlockSpec((1, ID), lambda i: (0, 0)),
            pl.BlockSpec((1, ID), lambda i: (0, 0)),
            pl.BlockSpec((1, KVR), lambda i: (0, 0)),
        ],
        out_specs=[
            pl.BlockSpec((ID, BT), lambda i: (0, i)),
            pl.BlockSpec((BT, IH), lambda i: (i, 0)),
            pl.BlockSpec((BT, KVR + ROPE), lambda i: (i, 0)),
            pl.BlockSpec((KVR + ROPE, BT), lambda i: (0, i)),
        ],
        out_shape=[
            jax.ShapeDtypeStruct((ID, T), jnp.bfloat16),
            jax.ShapeDtypeStruct((T, IH), f32),
            jax.ShapeDtypeStruct((T, KVR + ROPE), jnp.bfloat16),
            jax.ShapeDtypeStruct((KVR + ROPE, T), jnp.bfloat16),
        ],
    )(hidden_states, kv_c, k_pe, cos, sin, Wk_idx, Wweights, knw, knb, kvln)

    us, thr = pl.pallas_call(
        _indexer_kernel,
        grid=(nb,),
        in_specs=[
            pl.BlockSpec((BT, QL), lambda i: (i, 0)),
            pl.BlockSpec((QL, IH * ID), lambda i: (0, 0)),
            pl.BlockSpec((BT, ROPE // 2), lambda i: (i, 0)),
            pl.BlockSpec((BT, ROPE // 2), lambda i: (i, 0)),
            pl.BlockSpec((ID, T), lambda i: (0, 0)),
            pl.BlockSpec((BT, IH), lambda i: (i, 0)),
        ],
        out_specs=[
            pl.BlockSpec((BT, T), lambda i: (i, 0)),
            pl.BlockSpec((BT, 1), lambda i: (i, 0)),
        ],
        out_shape=[
            jax.ShapeDtypeStruct((T, T), jnp.uint32),
            jax.ShapeDtypeStruct((T, 1), jnp.uint32),
        ],
        scratch_shapes=[pltpu.VMEM((IH * BT, T), jnp.bfloat16)],
    )(q_c, Wq_idx, cos, sin, kiT, wts)

    q_full = pl.pallas_call(
        _qproj_kernel,
        grid=(nb, H),
        in_specs=[
            pl.BlockSpec((BT, QL), lambda i, h: (i, 0)),
            pl.BlockSpec((1, QL, NOPE + ROPE), lambda i, h: (h, 0, 0)),
            pl.BlockSpec((1, NOPE, KVR), lambda i, h: (h, 0, 0)),
            pl.BlockSpec((BT, ROPE // 2), lambda i, h: (i, 0)),
            pl.BlockSpec((BT, ROPE // 2), lambda i, h: (i, 0)),
        ],
        out_specs=pl.BlockSpec((1, BT, KVR + ROPE), lambda i, h: (h, i, 0)),
        out_shape=jax.ShapeDtypeStruct((H, T, KVR + ROPE), jnp.bfloat16),
    )(q_c, Wq_b_r, W_UK_T, cos, sin)

    v = pl.pallas_call(
        _flash_kernel,
        grid=(T // BTQ, T // BTK),
        in_specs=[
            pl.BlockSpec((H, BTQ, KVR + ROPE), lambda i, k: (0, i, 0)),
            pl.BlockSpec((KVR + ROPE, BTK), lambda i, k: (0, k)),
            pl.BlockSpec((BTK, KVR + ROPE), lambda i, k: (k, 0)),
            pl.BlockSpec((BTQ, BTK), lambda i, k: (i, k)),
            pl.BlockSpec((BTQ, 1), lambda i, k: (i, 0)),
            pl.BlockSpec((H, KVR, VD), lambda i, k: (0, 0, 0)),
        ],
        out_specs=pl.BlockSpec((BTQ, H * VD), lambda i, k: (i, 0)),
        out_shape=jax.ShapeDtypeStruct((T, H * VD), f32),
        scratch_shapes=[
            pltpu.VMEM((H, BTQ, KVR), f32),
            pltpu.VMEM((H, BTQ, 1), f32),
            pltpu.VMEM((H, BTQ, 1), f32),
        ],
    )(q_full, k_fullT, k_full, us, thr, W_UVb)

    out = pl.pallas_call(
        _oproj_kernel,
        grid=(nb,),
        in_specs=[
            pl.BlockSpec((BT, H * VD), lambda i: (i, 0)),
            pl.BlockSpec((H * VD, HID), lambda i: (0, 0)),
        ],
        out_specs=pl.BlockSpec((BT, HID), lambda i: (i, 0)),
        out_shape=jax.ShapeDtypeStruct((T, HID), f32),
    )(v, Wo)

    return out
